# baseline (speedup 1.0000x reference)
import functools
import math

import jax
import jax.numpy as jnp
from jax import lax
from jax.experimental import pallas as pl
from jax.experimental.pallas import tpu as pltpu
from jax.experimental.pallas import tpu_sc as plsc

_INV_SQRT_PI = 1.0 / math.sqrt(math.pi)
_CUT2 = 9.0

_ROWS = 8192
_COLS = 4096

_NC = 2
_NS = 16
_NW = _NC * _NS
_LANES = 16
_SC_CH = 4
_UNROLL = 8


def _sc_compute(bm_v, bv_v, c_v, o_v):
    for r in range(_SC_CH):
        c_b = c_v[r, :]
        nc2 = -(c_b * c_b)
        k_out = c_b * jnp.float32(_INV_SQRT_PI)
        k_cut = jnp.float32(_CUT2) / (c_b * c_b)

        def col_body(j, carry, r=r, nc2=nc2, k_out=k_out, k_cut=k_cut):
            base = j * (_UNROLL * _LANES)
            for u in range(_UNROLL):
                s0 = base + u * _LANES
                d = bm_v[r, pl.ds(s0, _LANES)] - bv_v[r, pl.ds(s0, _LANES)]
                s = d * d
                val = jnp.exp(s * nc2) * k_out
                o_v[r, pl.ds(s0, _LANES)] = jnp.where(
                    s <= k_cut, val, jnp.float32(0.0))
            return carry

        lax.fori_loop(0, _COLS // (_UNROLL * _LANES), col_body, 0)


def _make_sc(rows):
    rpw = rows // _NW
    nchunks = rpw // _SC_CH
    npairs = nchunks // 2
    mesh = plsc.VectorSubcoreMesh(core_axis_name="c", subcore_axis_name="s")

    buf = lambda: pltpu.VMEM((_SC_CH, _COLS), jnp.float32)
    cbuf = lambda: pltpu.VMEM((_SC_CH, _LANES), jnp.float32)

    @functools.partial(
        pl.kernel,
        mesh=mesh,
        out_type=jax.ShapeDtypeStruct((rows, _COLS), jnp.float32),
        scratch_types=[
            buf(), buf(), cbuf(), buf(),
            buf(), buf(), cbuf(), buf(),
            pltpu.SemaphoreType.DMA,
            pltpu.SemaphoreType.DMA,
            pltpu.SemaphoreType.DMA,
            pltpu.SemaphoreType.DMA,
        ],
    )
    def sc_kernel(bm_hbm, c_hbm, bv_hbm, out_hbm,
                  bm_a, bv_a, c_a, o_a, bm_b, bv_b, c_b, o_b,
                  sem_ia, sem_ib, sem_oa, sem_ob):
        wid = lax.axis_index("s") * _NC + lax.axis_index("c")
        base_row = wid * rpw

        def start_in(chunk, bm_v, bv_v, c_v, sem):
            row0 = base_row + chunk * _SC_CH
            pltpu.async_copy(bm_hbm.at[pl.ds(row0, _SC_CH)], bm_v, sem)
            pltpu.async_copy(bv_hbm.at[pl.ds(row0, _SC_CH)], bv_v, sem)
            pltpu.async_copy(c_hbm.at[pl.ds(row0, _SC_CH)], c_v, sem)

        def wait_in(bm_v, bv_v, c_v, sem):
            pltpu.make_async_copy(bm_hbm.at[pl.ds(base_row, _SC_CH)], bm_v,
                                  sem).wait()
            pltpu.make_async_copy(bv_hbm.at[pl.ds(base_row, _SC_CH)], bv_v,
                                  sem).wait()
            pltpu.make_async_copy(c_hbm.at[pl.ds(base_row, _SC_CH)], c_v,
                                  sem).wait()

        def start_out(chunk, o_v, sem):
            row0 = base_row + chunk * _SC_CH
            pltpu.async_copy(o_v, out_hbm.at[pl.ds(row0, _SC_CH)], sem)

        def wait_out(o_v, sem):
            pltpu.make_async_copy(o_v, out_hbm.at[pl.ds(base_row, _SC_CH)],
                                  sem).wait()

        start_in(0, bm_a, bv_a, c_a, sem_ia)

        def pair_body(g, carry):
            ch0 = 2 * g
            start_in(ch0 + 1, bm_b, bv_b, c_b, sem_ib)
            wait_in(bm_a, bv_a, c_a, sem_ia)

            @pl.when(g > 0)
            def _():
                wait_out(o_a, sem_oa)

            _sc_compute(bm_a, bv_a, c_a, o_a)
            start_out(ch0, o_a, sem_oa)

            @pl.when(g < npairs - 1)
            def _():
                start_in(ch0 + 2, bm_a, bv_a, c_a, sem_ia)

            wait_in(bm_b, bv_b, c_b, sem_ib)

            @pl.when(g > 0)
            def _():
                wait_out(o_b, sem_ob)

            _sc_compute(bm_b, bv_b, c_b, o_b)
            start_out(ch0 + 1, o_b, sem_ob)
            return carry

        lax.fori_loop(0, npairs, pair_body, 0)
        wait_out(o_a, sem_oa)
        wait_out(o_b, sem_ob)

    return sc_kernel


_TC_BLOCK_ROWS = 256


_TC_TILE_R = 8
_TC_TILE_C = 2048


def _tc_body(bm_ref, c_ref, bv_ref, o_ref):
    n_r = _TC_BLOCK_ROWS // _TC_TILE_R
    n_c = _COLS // _TC_TILE_C

    def tile_body(t, carry):
        r0 = (t // n_c) * _TC_TILE_R
        c0 = (t % n_c) * _TC_TILE_C
        c = c_ref[pl.ds(r0, _TC_TILE_R), :]
        bm = bm_ref[pl.ds(r0, _TC_TILE_R), pl.ds(c0, _TC_TILE_C)]
        bv = bv_ref[pl.ds(r0, _TC_TILE_R), pl.ds(c0, _TC_TILE_C)]
        arg = (bm - bv) * c
        absorption = jnp.exp(-(arg * arg)) * (c * jnp.float32(_INV_SQRT_PI))
        o_ref[pl.ds(r0, _TC_TILE_R), pl.ds(c0, _TC_TILE_C)] = jnp.where(
            jnp.abs(arg) <= jnp.float32(3.0), absorption, jnp.float32(0.0))
        return carry

    lax.fori_loop(0, n_r * n_c, tile_body, 0)


def _tc_call(B_mean, c_extended, B_val, row_start, n_rows):
    blk0 = row_start // _TC_BLOCK_ROWS
    grid = (n_rows // _TC_BLOCK_ROWS,)
    return pl.pallas_call(
        _tc_body,
        grid=grid,
        in_specs=[
            pl.BlockSpec((_TC_BLOCK_ROWS, _COLS), lambda i: (i + blk0, 0)),
            pl.BlockSpec((_TC_BLOCK_ROWS, 1), lambda i: (i + blk0, 0)),
            pl.BlockSpec((_TC_BLOCK_ROWS, _COLS), lambda i: (i + blk0, 0)),
        ],
        out_specs=pl.BlockSpec((_TC_BLOCK_ROWS, _COLS),
                               lambda i: (i + blk0, 0)),
        out_shape=jax.ShapeDtypeStruct((_ROWS, _COLS), jnp.float32),
    )(B_mean, c_extended, B_val)


@jax.jit
def kernel(B_mean, c_extended, B_val):
    return _tc_call(B_mean, c_extended, B_val, 0, _ROWS)


# speedup vs baseline: 1.0547x; 1.0547x over previous
import functools
import math

import jax
import jax.numpy as jnp
from jax import lax
from jax.experimental import pallas as pl
from jax.experimental.pallas import tpu as pltpu
from jax.experimental.pallas import tpu_sc as plsc

_INV_SQRT_PI = 1.0 / math.sqrt(math.pi)
_CUT2 = 9.0

_ROWS = 8192
_COLS = 4096

_NC = 2
_NS = 16
_NW = _NC * _NS
_LANES = 16
_SC_CH = 4
_UNROLL = 8


def _sc_compute(bm_v, bv_v, c_v, o_v):
    for r in range(_SC_CH):
        c_b = c_v[r, :]
        nc2 = -(c_b * c_b)
        k_out = c_b * jnp.float32(_INV_SQRT_PI)
        k_cut = jnp.float32(_CUT2) / (c_b * c_b)

        def col_body(j, carry, r=r, nc2=nc2, k_out=k_out, k_cut=k_cut):
            base = j * (_UNROLL * _LANES)
            for u in range(_UNROLL):
                s0 = base + u * _LANES
                d = bm_v[r, pl.ds(s0, _LANES)] - bv_v[r, pl.ds(s0, _LANES)]
                s = d * d
                val = jnp.exp(s * nc2) * k_out
                o_v[r, pl.ds(s0, _LANES)] = jnp.where(
                    s <= k_cut, val, jnp.float32(0.0))
            return carry

        lax.fori_loop(0, _COLS // (_UNROLL * _LANES), col_body, 0)


def _make_sc(rows):
    rpw = rows // _NW
    nchunks = rpw // _SC_CH
    npairs = nchunks // 2
    mesh = plsc.VectorSubcoreMesh(core_axis_name="c", subcore_axis_name="s")

    buf = lambda: pltpu.VMEM((_SC_CH, _COLS), jnp.float32)
    cbuf = lambda: pltpu.VMEM((_SC_CH, _LANES), jnp.float32)

    @functools.partial(
        pl.kernel,
        mesh=mesh,
        out_type=jax.ShapeDtypeStruct((rows, _COLS), jnp.float32),
        scratch_types=[
            buf(), buf(), cbuf(), buf(),
            buf(), buf(), cbuf(), buf(),
            pltpu.SemaphoreType.DMA,
            pltpu.SemaphoreType.DMA,
            pltpu.SemaphoreType.DMA,
            pltpu.SemaphoreType.DMA,
        ],
    )
    def sc_kernel(bm_hbm, c_hbm, bv_hbm, out_hbm,
                  bm_a, bv_a, c_a, o_a, bm_b, bv_b, c_b, o_b,
                  sem_ia, sem_ib, sem_oa, sem_ob):
        wid = lax.axis_index("s") * _NC + lax.axis_index("c")
        base_row = wid * rpw

        def start_in(chunk, bm_v, bv_v, c_v, sem):
            row0 = base_row + chunk * _SC_CH
            pltpu.async_copy(bm_hbm.at[pl.ds(row0, _SC_CH)], bm_v, sem)
            pltpu.async_copy(bv_hbm.at[pl.ds(row0, _SC_CH)], bv_v, sem)
            pltpu.async_copy(c_hbm.at[pl.ds(row0, _SC_CH)], c_v, sem)

        def wait_in(bm_v, bv_v, c_v, sem):
            pltpu.make_async_copy(bm_hbm.at[pl.ds(base_row, _SC_CH)], bm_v,
                                  sem).wait()
            pltpu.make_async_copy(bv_hbm.at[pl.ds(base_row, _SC_CH)], bv_v,
                                  sem).wait()
            pltpu.make_async_copy(c_hbm.at[pl.ds(base_row, _SC_CH)], c_v,
                                  sem).wait()

        def start_out(chunk, o_v, sem):
            row0 = base_row + chunk * _SC_CH
            pltpu.async_copy(o_v, out_hbm.at[pl.ds(row0, _SC_CH)], sem)

        def wait_out(o_v, sem):
            pltpu.make_async_copy(o_v, out_hbm.at[pl.ds(base_row, _SC_CH)],
                                  sem).wait()

        start_in(0, bm_a, bv_a, c_a, sem_ia)

        def pair_body(g, carry):
            ch0 = 2 * g
            start_in(ch0 + 1, bm_b, bv_b, c_b, sem_ib)
            wait_in(bm_a, bv_a, c_a, sem_ia)

            @pl.when(g > 0)
            def _():
                wait_out(o_a, sem_oa)

            _sc_compute(bm_a, bv_a, c_a, o_a)
            start_out(ch0, o_a, sem_oa)

            @pl.when(g < npairs - 1)
            def _():
                start_in(ch0 + 2, bm_a, bv_a, c_a, sem_ia)

            wait_in(bm_b, bv_b, c_b, sem_ib)

            @pl.when(g > 0)
            def _():
                wait_out(o_b, sem_ob)

            _sc_compute(bm_b, bv_b, c_b, o_b)
            start_out(ch0 + 1, o_b, sem_ob)
            return carry

        lax.fori_loop(0, npairs, pair_body, 0)
        wait_out(o_a, sem_oa)
        wait_out(o_b, sem_ob)

    return sc_kernel


_TC_BLOCK_ROWS = 256


def _tc_body(bm_ref, c_ref, bv_ref, o_ref):
    c = c_ref[...]
    arg = (bm_ref[...] - bv_ref[...]) * c
    absorption = jnp.exp(-(arg * arg)) * (c * jnp.float32(_INV_SQRT_PI))
    o_ref[...] = jnp.where(jnp.abs(arg) <= jnp.float32(3.0), absorption,
                           jnp.float32(0.0))


def _tc_call(B_mean, c_extended, B_val, row_start, n_rows):
    blk0 = row_start // _TC_BLOCK_ROWS
    grid = (n_rows // _TC_BLOCK_ROWS,)
    return pl.pallas_call(
        _tc_body,
        grid=grid,
        in_specs=[
            pl.BlockSpec((_TC_BLOCK_ROWS, _COLS), lambda i: (i + blk0, 0)),
            pl.BlockSpec((_TC_BLOCK_ROWS, 1), lambda i: (i + blk0, 0)),
            pl.BlockSpec((_TC_BLOCK_ROWS, _COLS), lambda i: (i + blk0, 0)),
        ],
        out_specs=pl.BlockSpec((_TC_BLOCK_ROWS, _COLS),
                               lambda i: (i + blk0, 0)),
        out_shape=jax.ShapeDtypeStruct((_ROWS, _COLS), jnp.float32),
    )(B_mean, c_extended, B_val)


_M_CH = 64


def _tc_manual_body(bm_hbm, c_hbm, bv_hbm, o_hbm,
                    c_all, bm_a, bv_a, o_a, bm_b, bv_b, o_b,
                    sem_c, sem_ia, sem_ib, sem_oa, sem_ob):
    nchunks = _ROWS // _M_CH
    npairs = nchunks // 2

    cp_c = pltpu.make_async_copy(c_hbm, c_all, sem_c)
    cp_c.start()

    def start_in(chunk, bm_v, bv_v, sem):
        r0 = chunk * _M_CH
        pltpu.make_async_copy(bm_hbm.at[pl.ds(r0, _M_CH)], bm_v, sem).start()
        pltpu.make_async_copy(bv_hbm.at[pl.ds(r0, _M_CH)], bv_v, sem).start()

    def wait_in(bm_v, bv_v, sem):
        pltpu.make_async_copy(bm_hbm.at[pl.ds(0, _M_CH)], bm_v, sem).wait()
        pltpu.make_async_copy(bv_hbm.at[pl.ds(0, _M_CH)], bv_v, sem).wait()

    def start_out(chunk, o_v, sem):
        r0 = chunk * _M_CH
        pltpu.make_async_copy(o_v, o_hbm.at[pl.ds(r0, _M_CH)], sem).start()

    def wait_out(o_v, sem):
        pltpu.make_async_copy(o_v, o_hbm.at[pl.ds(0, _M_CH)], sem).wait()

    def compute(chunk, bm_v, bv_v, o_v):
        r0 = chunk * _M_CH
        c = c_all[pl.ds(r0, _M_CH), :]
        arg = (bm_v[...] - bv_v[...]) * c
        absorption = jnp.exp(-(arg * arg)) * (c * jnp.float32(_INV_SQRT_PI))
        o_v[...] = jnp.where(jnp.abs(arg) <= jnp.float32(3.0), absorption,
                             jnp.float32(0.0))

    start_in(0, bm_a, bv_a, sem_ia)
    cp_c.wait()

    def pair_body(g, carry):
        ch0 = 2 * g
        start_in(ch0 + 1, bm_b, bv_b, sem_ib)
        wait_in(bm_a, bv_a, sem_ia)

        @pl.when(g > 0)
        def _():
            wait_out(o_a, sem_oa)

        compute(ch0, bm_a, bv_a, o_a)
        start_out(ch0, o_a, sem_oa)

        @pl.when(g < npairs - 1)
        def _():
            start_in(ch0 + 2, bm_a, bv_a, sem_ia)

        wait_in(bm_b, bv_b, sem_ib)

        @pl.when(g > 0)
        def _():
            wait_out(o_b, sem_ob)

        compute(ch0 + 1, bm_b, bv_b, o_b)
        start_out(ch0 + 1, o_b, sem_ob)
        return carry

    lax.fori_loop(0, npairs, pair_body, 0)
    wait_out(o_a, sem_oa)
    wait_out(o_b, sem_ob)


def _tc_manual(B_mean, c_extended, B_val):
    buf = lambda: pltpu.VMEM((_M_CH, _COLS), jnp.float32)
    return pl.pallas_call(
        _tc_manual_body,
        in_specs=[
            pl.BlockSpec(memory_space=pltpu.MemorySpace.HBM),
            pl.BlockSpec(memory_space=pltpu.MemorySpace.HBM),
            pl.BlockSpec(memory_space=pltpu.MemorySpace.HBM),
        ],
        out_specs=pl.BlockSpec(memory_space=pltpu.MemorySpace.HBM),
        out_shape=jax.ShapeDtypeStruct((_ROWS, _COLS), jnp.float32),
        scratch_shapes=[
            pltpu.VMEM((_ROWS, 1), jnp.float32),
            buf(), buf(), buf(), buf(), buf(), buf(),
            pltpu.SemaphoreType.DMA,
            pltpu.SemaphoreType.DMA,
            pltpu.SemaphoreType.DMA,
            pltpu.SemaphoreType.DMA,
            pltpu.SemaphoreType.DMA,
        ],
    )(B_mean, c_extended, B_val)


@jax.jit
def kernel(B_mean, c_extended, B_val):
    return _tc_manual(B_mean, c_extended, B_val)


# speedup vs baseline: 1.5495x; 1.4691x over previous
import functools
import math

import jax
import jax.numpy as jnp
from jax import lax
from jax.experimental import pallas as pl
from jax.experimental.pallas import tpu as pltpu
from jax.experimental.pallas import tpu_sc as plsc

_INV_SQRT_PI = 1.0 / math.sqrt(math.pi)
_CUT2 = 9.0

_ROWS = 8192
_COLS = 4096

_NC = 2
_NS = 16
_NW = _NC * _NS
_LANES = 16
_SC_CH = 4
_UNROLL = 8


def _sc_compute(bm_v, bv_v, c_v, o_v):
    for r in range(_SC_CH):
        c_b = c_v[r, :]
        nc2 = -(c_b * c_b)
        k_out = c_b * jnp.float32(_INV_SQRT_PI)
        k_cut = jnp.float32(_CUT2) / (c_b * c_b)

        def col_body(j, carry, r=r, nc2=nc2, k_out=k_out, k_cut=k_cut):
            base = j * (_UNROLL * _LANES)
            for u in range(_UNROLL):
                s0 = base + u * _LANES
                d = bm_v[r, pl.ds(s0, _LANES)] - bv_v[r, pl.ds(s0, _LANES)]
                s = d * d
                val = jnp.exp(s * nc2) * k_out
                o_v[r, pl.ds(s0, _LANES)] = jnp.where(
                    s <= k_cut, val, jnp.float32(0.0))
            return carry

        lax.fori_loop(0, _COLS // (_UNROLL * _LANES), col_body, 0)


def _make_sc(rows):
    rpw = rows // _NW
    nchunks = rpw // _SC_CH
    npairs = nchunks // 2
    mesh = plsc.VectorSubcoreMesh(core_axis_name="c", subcore_axis_name="s")

    buf = lambda: pltpu.VMEM((_SC_CH, _COLS), jnp.float32)
    cbuf = lambda: pltpu.VMEM((_SC_CH, _LANES), jnp.float32)

    @functools.partial(
        pl.kernel,
        mesh=mesh,
        out_type=jax.ShapeDtypeStruct((rows, _COLS), jnp.float32),
        scratch_types=[
            buf(), buf(), cbuf(), buf(),
            buf(), buf(), cbuf(), buf(),
            pltpu.SemaphoreType.DMA,
            pltpu.SemaphoreType.DMA,
            pltpu.SemaphoreType.DMA,
            pltpu.SemaphoreType.DMA,
        ],
    )
    def sc_kernel(bm_hbm, c_hbm, bv_hbm, out_hbm,
                  bm_a, bv_a, c_a, o_a, bm_b, bv_b, c_b, o_b,
                  sem_ia, sem_ib, sem_oa, sem_ob):
        wid = lax.axis_index("s") * _NC + lax.axis_index("c")
        base_row = wid * rpw

        def start_in(chunk, bm_v, bv_v, c_v, sem):
            row0 = base_row + chunk * _SC_CH
            pltpu.async_copy(bm_hbm.at[pl.ds(row0, _SC_CH)], bm_v, sem)
            pltpu.async_copy(bv_hbm.at[pl.ds(row0, _SC_CH)], bv_v, sem)
            pltpu.async_copy(c_hbm.at[pl.ds(row0, _SC_CH)], c_v, sem)

        def wait_in(bm_v, bv_v, c_v, sem):
            pltpu.make_async_copy(bm_hbm.at[pl.ds(base_row, _SC_CH)], bm_v,
                                  sem).wait()
            pltpu.make_async_copy(bv_hbm.at[pl.ds(base_row, _SC_CH)], bv_v,
                                  sem).wait()
            pltpu.make_async_copy(c_hbm.at[pl.ds(base_row, _SC_CH)], c_v,
                                  sem).wait()

        def start_out(chunk, o_v, sem):
            row0 = base_row + chunk * _SC_CH
            pltpu.async_copy(o_v, out_hbm.at[pl.ds(row0, _SC_CH)], sem)

        def wait_out(o_v, sem):
            pltpu.make_async_copy(o_v, out_hbm.at[pl.ds(base_row, _SC_CH)],
                                  sem).wait()

        start_in(0, bm_a, bv_a, c_a, sem_ia)

        def pair_body(g, carry):
            ch0 = 2 * g
            start_in(ch0 + 1, bm_b, bv_b, c_b, sem_ib)
            wait_in(bm_a, bv_a, c_a, sem_ia)

            @pl.when(g > 0)
            def _():
                wait_out(o_a, sem_oa)

            _sc_compute(bm_a, bv_a, c_a, o_a)
            start_out(ch0, o_a, sem_oa)

            @pl.when(g < npairs - 1)
            def _():
                start_in(ch0 + 2, bm_a, bv_a, c_a, sem_ia)

            wait_in(bm_b, bv_b, c_b, sem_ib)

            @pl.when(g > 0)
            def _():
                wait_out(o_b, sem_ob)

            _sc_compute(bm_b, bv_b, c_b, o_b)
            start_out(ch0 + 1, o_b, sem_ob)
            return carry

        lax.fori_loop(0, npairs, pair_body, 0)
        wait_out(o_a, sem_oa)
        wait_out(o_b, sem_ob)

    return sc_kernel


_TC_BLOCK_ROWS = 256


def _tc_body(bm_ref, c_ref, bv_ref, o_ref):
    c = c_ref[...]
    arg = (bm_ref[...] - bv_ref[...]) * c
    absorption = jnp.exp(-(arg * arg)) * (c * jnp.float32(_INV_SQRT_PI))
    o_ref[...] = jnp.where(jnp.abs(arg) <= jnp.float32(3.0), absorption,
                           jnp.float32(0.0))


def _tc_call(B_mean, c_extended, B_val, row_start, n_rows):
    blk0 = row_start // _TC_BLOCK_ROWS
    grid = (n_rows // _TC_BLOCK_ROWS,)
    return pl.pallas_call(
        _tc_body,
        grid=grid,
        in_specs=[
            pl.BlockSpec((_TC_BLOCK_ROWS, _COLS), lambda i: (i + blk0, 0)),
            pl.BlockSpec((_TC_BLOCK_ROWS, 1), lambda i: (i + blk0, 0)),
            pl.BlockSpec((_TC_BLOCK_ROWS, _COLS), lambda i: (i + blk0, 0)),
        ],
        out_specs=pl.BlockSpec((_TC_BLOCK_ROWS, _COLS),
                               lambda i: (i + blk0, 0)),
        out_shape=jax.ShapeDtypeStruct((_ROWS, _COLS), jnp.float32),
    )(B_mean, c_extended, B_val)


_M_CH = 64


_NBUF = 4


def _tc_manual_body(bm_hbm, c_hbm, bv_hbm, o_hbm, c_all, *rest):
    bufs = rest[:3 * _NBUF]
    sems = rest[3 * _NBUF:]
    sem_c = sems[0]
    sem_in = sems[1:1 + _NBUF]
    sem_out = sems[1 + _NBUF:1 + 2 * _NBUF]
    bm_bufs = bufs[0::3]
    bv_bufs = bufs[1::3]
    o_bufs = bufs[2::3]

    nchunks = _ROWS // _M_CH
    ngroups = nchunks // _NBUF

    cp_c = pltpu.make_async_copy(c_hbm, c_all, sem_c)
    cp_c.start()

    def start_in(chunk, s):
        r0 = chunk * _M_CH
        pltpu.make_async_copy(bm_hbm.at[pl.ds(r0, _M_CH)], bm_bufs[s],
                              sem_in[s]).start()
        pltpu.make_async_copy(bv_hbm.at[pl.ds(r0, _M_CH)], bv_bufs[s],
                              sem_in[s]).start()

    def wait_in(s):
        pltpu.make_async_copy(bm_hbm.at[pl.ds(0, _M_CH)], bm_bufs[s],
                              sem_in[s]).wait()
        pltpu.make_async_copy(bv_hbm.at[pl.ds(0, _M_CH)], bv_bufs[s],
                              sem_in[s]).wait()

    def start_out(chunk, s):
        r0 = chunk * _M_CH
        pltpu.make_async_copy(o_bufs[s], o_hbm.at[pl.ds(r0, _M_CH)],
                              sem_out[s]).start()

    def wait_out(s):
        pltpu.make_async_copy(o_bufs[s], o_hbm.at[pl.ds(0, _M_CH)],
                              sem_out[s]).wait()

    def compute(chunk, s):
        r0 = chunk * _M_CH
        c = c_all[pl.ds(r0, _M_CH), :]
        arg = (bm_bufs[s][...] - bv_bufs[s][...]) * c
        absorption = jnp.exp(-(arg * arg)) * (c * jnp.float32(_INV_SQRT_PI))
        o_bufs[s][...] = jnp.where(jnp.abs(arg) <= jnp.float32(3.0),
                                   absorption, jnp.float32(0.0))

    for s in range(_NBUF - 1):
        start_in(s, s)
    cp_c.wait()

    def group_body(g, carry):
        ch0 = g * _NBUF
        for s in range(_NBUF):
            chunk = ch0 + s

            @pl.when(chunk + _NBUF - 1 < nchunks)
            def _(chunk=chunk, s=s):
                start_in(chunk + _NBUF - 1, (s + _NBUF - 1) % _NBUF)

            wait_in(s)

            @pl.when(g > 0)
            def _(s=s):
                wait_out(s)

            compute(chunk, s)
            start_out(chunk, s)
        return carry

    lax.fori_loop(0, ngroups, group_body, 0)
    for s in range(_NBUF):
        wait_out(s)


def _tc_manual(B_mean, c_extended, B_val):
    buf = lambda: pltpu.VMEM((_M_CH, _COLS), jnp.float32)
    return pl.pallas_call(
        _tc_manual_body,
        in_specs=[
            pl.BlockSpec(memory_space=pltpu.MemorySpace.HBM),
            pl.BlockSpec(memory_space=pltpu.MemorySpace.HBM),
            pl.BlockSpec(memory_space=pltpu.MemorySpace.HBM),
        ],
        out_specs=pl.BlockSpec(memory_space=pltpu.MemorySpace.HBM),
        out_shape=jax.ShapeDtypeStruct((_ROWS, _COLS), jnp.float32),
        scratch_shapes=(
            [pltpu.VMEM((_ROWS, 1), jnp.float32)]
            + [buf() for _ in range(3 * _NBUF)]
            + [pltpu.SemaphoreType.DMA] * (1 + 2 * _NBUF)
        ),
    )(B_mean, c_extended, B_val)


@jax.jit
def kernel(B_mean, c_extended, B_val):
    return _tc_manual(B_mean, c_extended, B_val)


# speedup vs baseline: 1.5573x; 1.0051x over previous
import functools
import math

import jax
import jax.numpy as jnp
from jax import lax
from jax.experimental import pallas as pl
from jax.experimental.pallas import tpu as pltpu
from jax.experimental.pallas import tpu_sc as plsc

_INV_SQRT_PI = 1.0 / math.sqrt(math.pi)
_CUT2 = 9.0

_ROWS = 8192
_COLS = 4096

_NC = 2
_NS = 16
_NW = _NC * _NS
_LANES = 16
_SC_CH = 4
_UNROLL = 8


def _sc_compute(bm_v, bv_v, c_v, o_v):
    for r in range(_SC_CH):
        c_b = c_v[r, :]
        nc2 = -(c_b * c_b)
        k_out = c_b * jnp.float32(_INV_SQRT_PI)
        k_cut = jnp.float32(_CUT2) / (c_b * c_b)

        def col_body(j, carry, r=r, nc2=nc2, k_out=k_out, k_cut=k_cut):
            base = j * (_UNROLL * _LANES)
            for u in range(_UNROLL):
                s0 = base + u * _LANES
                d = bm_v[r, pl.ds(s0, _LANES)] - bv_v[r, pl.ds(s0, _LANES)]
                s = d * d
                val = jnp.exp(s * nc2) * k_out
                o_v[r, pl.ds(s0, _LANES)] = jnp.where(
                    s <= k_cut, val, jnp.float32(0.0))
            return carry

        lax.fori_loop(0, _COLS // (_UNROLL * _LANES), col_body, 0)


def _make_sc(rows):
    rpw = rows // _NW
    nchunks = rpw // _SC_CH
    npairs = nchunks // 2
    mesh = plsc.VectorSubcoreMesh(core_axis_name="c", subcore_axis_name="s")

    buf = lambda: pltpu.VMEM((_SC_CH, _COLS), jnp.float32)
    cbuf = lambda: pltpu.VMEM((_SC_CH, _LANES), jnp.float32)

    @functools.partial(
        pl.kernel,
        mesh=mesh,
        out_type=jax.ShapeDtypeStruct((rows, _COLS), jnp.float32),
        scratch_types=[
            buf(), buf(), cbuf(), buf(),
            buf(), buf(), cbuf(), buf(),
            pltpu.SemaphoreType.DMA,
            pltpu.SemaphoreType.DMA,
            pltpu.SemaphoreType.DMA,
            pltpu.SemaphoreType.DMA,
        ],
    )
    def sc_kernel(bm_hbm, c_hbm, bv_hbm, out_hbm,
                  bm_a, bv_a, c_a, o_a, bm_b, bv_b, c_b, o_b,
                  sem_ia, sem_ib, sem_oa, sem_ob):
        wid = lax.axis_index("s") * _NC + lax.axis_index("c")
        base_row = wid * rpw

        def start_in(chunk, bm_v, bv_v, c_v, sem):
            row0 = base_row + chunk * _SC_CH
            pltpu.async_copy(bm_hbm.at[pl.ds(row0, _SC_CH)], bm_v, sem)
            pltpu.async_copy(bv_hbm.at[pl.ds(row0, _SC_CH)], bv_v, sem)
            pltpu.async_copy(c_hbm.at[pl.ds(row0, _SC_CH)], c_v, sem)

        def wait_in(bm_v, bv_v, c_v, sem):
            pltpu.make_async_copy(bm_hbm.at[pl.ds(base_row, _SC_CH)], bm_v,
                                  sem).wait()
            pltpu.make_async_copy(bv_hbm.at[pl.ds(base_row, _SC_CH)], bv_v,
                                  sem).wait()
            pltpu.make_async_copy(c_hbm.at[pl.ds(base_row, _SC_CH)], c_v,
                                  sem).wait()

        def start_out(chunk, o_v, sem):
            row0 = base_row + chunk * _SC_CH
            pltpu.async_copy(o_v, out_hbm.at[pl.ds(row0, _SC_CH)], sem)

        def wait_out(o_v, sem):
            pltpu.make_async_copy(o_v, out_hbm.at[pl.ds(base_row, _SC_CH)],
                                  sem).wait()

        start_in(0, bm_a, bv_a, c_a, sem_ia)

        def pair_body(g, carry):
            ch0 = 2 * g
            start_in(ch0 + 1, bm_b, bv_b, c_b, sem_ib)
            wait_in(bm_a, bv_a, c_a, sem_ia)

            @pl.when(g > 0)
            def _():
                wait_out(o_a, sem_oa)

            _sc_compute(bm_a, bv_a, c_a, o_a)
            start_out(ch0, o_a, sem_oa)

            @pl.when(g < npairs - 1)
            def _():
                start_in(ch0 + 2, bm_a, bv_a, c_a, sem_ia)

            wait_in(bm_b, bv_b, c_b, sem_ib)

            @pl.when(g > 0)
            def _():
                wait_out(o_b, sem_ob)

            _sc_compute(bm_b, bv_b, c_b, o_b)
            start_out(ch0 + 1, o_b, sem_ob)
            return carry

        lax.fori_loop(0, npairs, pair_body, 0)
        wait_out(o_a, sem_oa)
        wait_out(o_b, sem_ob)

    return sc_kernel


_TC_BLOCK_ROWS = 256


def _tc_body(bm_ref, c_ref, bv_ref, o_ref):
    c = c_ref[...]
    arg = (bm_ref[...] - bv_ref[...]) * c
    absorption = jnp.exp(-(arg * arg)) * (c * jnp.float32(_INV_SQRT_PI))
    o_ref[...] = jnp.where(jnp.abs(arg) <= jnp.float32(3.0), absorption,
                           jnp.float32(0.0))


def _tc_call(B_mean, c_extended, B_val, row_start, n_rows):
    blk0 = row_start // _TC_BLOCK_ROWS
    grid = (n_rows // _TC_BLOCK_ROWS,)
    return pl.pallas_call(
        _tc_body,
        grid=grid,
        in_specs=[
            pl.BlockSpec((_TC_BLOCK_ROWS, _COLS), lambda i: (i + blk0, 0)),
            pl.BlockSpec((_TC_BLOCK_ROWS, 1), lambda i: (i + blk0, 0)),
            pl.BlockSpec((_TC_BLOCK_ROWS, _COLS), lambda i: (i + blk0, 0)),
        ],
        out_specs=pl.BlockSpec((_TC_BLOCK_ROWS, _COLS),
                               lambda i: (i + blk0, 0)),
        out_shape=jax.ShapeDtypeStruct((_ROWS, _COLS), jnp.float32),
    )(B_mean, c_extended, B_val)


_M_CH = 64


_NBUF = 8


def _tc_manual_body(bm_hbm, c_hbm, bv_hbm, o_hbm, c_all, *rest):
    bufs = rest[:3 * _NBUF]
    sems = rest[3 * _NBUF:]
    sem_c = sems[0]
    sem_in = sems[1:1 + _NBUF]
    sem_out = sems[1 + _NBUF:1 + 2 * _NBUF]
    bm_bufs = bufs[0::3]
    bv_bufs = bufs[1::3]
    o_bufs = bufs[2::3]

    nchunks = _ROWS // _M_CH
    ngroups = nchunks // _NBUF

    cp_c = pltpu.make_async_copy(c_hbm, c_all, sem_c)
    cp_c.start()

    def start_in(chunk, s):
        r0 = chunk * _M_CH
        pltpu.make_async_copy(bm_hbm.at[pl.ds(r0, _M_CH)], bm_bufs[s],
                              sem_in[s]).start()
        pltpu.make_async_copy(bv_hbm.at[pl.ds(r0, _M_CH)], bv_bufs[s],
                              sem_in[s]).start()

    def wait_in(s):
        pltpu.make_async_copy(bm_hbm.at[pl.ds(0, _M_CH)], bm_bufs[s],
                              sem_in[s]).wait()
        pltpu.make_async_copy(bv_hbm.at[pl.ds(0, _M_CH)], bv_bufs[s],
                              sem_in[s]).wait()

    def start_out(chunk, s):
        r0 = chunk * _M_CH
        pltpu.make_async_copy(o_bufs[s], o_hbm.at[pl.ds(r0, _M_CH)],
                              sem_out[s]).start()

    def wait_out(s):
        pltpu.make_async_copy(o_bufs[s], o_hbm.at[pl.ds(0, _M_CH)],
                              sem_out[s]).wait()

    def compute(chunk, s):
        r0 = chunk * _M_CH
        c = c_all[pl.ds(r0, _M_CH), :]
        arg = (bm_bufs[s][...] - bv_bufs[s][...]) * c
        absorption = jnp.exp(-(arg * arg)) * (c * jnp.float32(_INV_SQRT_PI))
        o_bufs[s][...] = jnp.where(jnp.abs(arg) <= jnp.float32(3.0),
                                   absorption, jnp.float32(0.0))

    for s in range(_NBUF - 1):
        start_in(s, s)
    cp_c.wait()

    def group_body(g, carry):
        ch0 = g * _NBUF
        for s in range(_NBUF):
            chunk = ch0 + s

            @pl.when(chunk + _NBUF - 1 < nchunks)
            def _(chunk=chunk, s=s):
                start_in(chunk + _NBUF - 1, (s + _NBUF - 1) % _NBUF)

            wait_in(s)

            @pl.when(g > 0)
            def _(s=s):
                wait_out(s)

            compute(chunk, s)
            start_out(chunk, s)
        return carry

    lax.fori_loop(0, ngroups, group_body, 0)
    for s in range(_NBUF):
        wait_out(s)


def _tc_manual(B_mean, c_extended, B_val):
    buf = lambda: pltpu.VMEM((_M_CH, _COLS), jnp.float32)
    return pl.pallas_call(
        _tc_manual_body,
        in_specs=[
            pl.BlockSpec(memory_space=pltpu.MemorySpace.HBM),
            pl.BlockSpec(memory_space=pltpu.MemorySpace.HBM),
            pl.BlockSpec(memory_space=pltpu.MemorySpace.HBM),
        ],
        out_specs=pl.BlockSpec(memory_space=pltpu.MemorySpace.HBM),
        out_shape=jax.ShapeDtypeStruct((_ROWS, _COLS), jnp.float32),
        scratch_shapes=(
            [pltpu.VMEM((_ROWS, 1), jnp.float32)]
            + [buf() for _ in range(3 * _NBUF)]
            + [pltpu.SemaphoreType.DMA] * (1 + 2 * _NBUF)
        ),
    )(B_mean, c_extended, B_val)


@jax.jit
def kernel(B_mean, c_extended, B_val):
    return _tc_manual(B_mean, c_extended, B_val)
